# baseline (speedup 1.0000x reference)
import functools

import jax
import jax.numpy as jnp
from jax import lax
from jax.experimental import pallas as pl
from jax.experimental.pallas import tpu as pltpu
from jax.experimental.pallas import tpu_sc as plsc

N = 10000
E = 320000
D = 128

NC = 2
NS = 16
NW = NC * NS

EPW = E // NW
CB = 64
NCHUNK = 160
GC = 8
NG = NCHUNK // GC
EPW_PAD = NCHUNK * CB

NA = N + 8

ROWS_PT = N // 10
CNT_PT = N // 10


def _matmul_body(x_ref, wa_ref, wb_ref, b_ref, t_out):
    xb = x_ref[...]
    t_out[:NA] = jnp.dot(xb, wa_ref[...], preferred_element_type=jnp.float32) + b_ref[...]
    t_out[NA:] = jnp.dot(xb, wb_ref[...], preferred_element_type=jnp.float32)


def _node_features(x_pad, wa, wb, b2d):
    return pl.pallas_call(
        _matmul_body,
        out_shape=jax.ShapeDtypeStruct((2 * NA, D), jnp.float32),
    )(x_pad, wa, wb, b2d)


def _edge_body(t_hbm, comb_hbm, dst_hbm, p_hbm, cnt0_hbm, cnt1_hbm,
               acc, cnt_s, comb_v, dst_v, g, m0, m1, ones_v, zcnt,
               sem_g, sem_s0, sem_s1, sem_c0, sem_c1):
    cid = lax.axis_index("c")
    sid = lax.axis_index("s")

    def _zero_m0(r, _):
        for k in range(8):
            m0[r, pl.ds(k * 16, 16)] = jnp.zeros((16,), jnp.float32)
        return ()
    lax.fori_loop(0, CB, _zero_m0, (), unroll=False)

    def _zero_zcnt(i, _):
        zcnt[pl.ds(i * 16, 16)] = jnp.zeros((16,), jnp.float32)
        return ()
    lax.fori_loop(0, 63, _zero_zcnt, (), unroll=False)

    @pl.when(sid < 10)
    def _():
        for j in range(ROWS_PT // CB):
            base = sid * ROWS_PT + j * CB
            pltpu.sync_copy(m0, acc.at[pl.ds(base, CB)])
        pltpu.sync_copy(m0.at[pl.ds(0, 40)],
                        acc.at[pl.ds(sid * ROWS_PT + (ROWS_PT // CB) * CB, 40)])
        pltpu.sync_copy(zcnt.at[pl.ds(0, CNT_PT)],
                        cnt_s.at[pl.ds(sid * CNT_PT, CNT_PT)])

    @pl.when(sid == 10)
    def _():
        pltpu.sync_copy(m0.at[pl.ds(0, 8)], acc.at[pl.ds(N, 8)])
        pltpu.sync_copy(zcnt.at[pl.ds(0, 8)], cnt_s.at[pl.ds(N, 8)])

    for k in range(CB // 16):
        ones_v[pl.ds(k * 16, 16)] = jnp.ones((16,), jnp.float32)

    plsc.subcore_barrier()

    wid = cid * NS + sid

    mbuf = (m0, m1)
    ssem = (sem_s0, sem_s1)
    csem = (sem_c0, sem_c1)

    def _drain(k):
        i = k % 2
        idx = dst_v.at[k]
        pltpu.make_async_copy(mbuf[i], acc.at[idx], ssem[i]).wait()
        pltpu.make_async_copy(ones_v, cnt_s.at[idx], csem[i]).wait()

    def _chunk(k):
        i = k % 2
        cp_g = pltpu.async_copy(t_hbm.at[comb_v.at[k]], g, sem_g)
        if k >= 2:
            _drain(k - 2)
        cp_g.wait()

        def _row(e, _):
            for kk in range(8):
                sl = pl.ds(kk * 16, 16)
                v = g[e, sl] + g[e + CB, sl]
                mbuf[i][e, sl] = jnp.maximum(v, 0.0)
            return ()
        lax.fori_loop(0, CB, _row, (), unroll=False)

        idx_d = dst_v.at[k]
        pltpu.async_copy(mbuf[i], acc.at[idx_d], ssem[i], add=True)
        pltpu.async_copy(ones_v, cnt_s.at[idx_d], csem[i], add=True)

    def _group(gi, _):
        pltpu.sync_copy(comb_hbm.at[wid, pl.ds(gi * GC, GC)], comb_v)
        pltpu.sync_copy(dst_hbm.at[wid, pl.ds(gi * GC, GC)], dst_v)
        for k in range(GC):
            _chunk(k)
        _drain(GC - 2)
        _drain(GC - 1)
        return ()
    lax.fori_loop(0, NG, _group, (), unroll=False)

    plsc.subcore_barrier()

    @pl.when(sid < 10)
    def _():
        for j in range(ROWS_PT // (2 * CB)):
            base = sid * ROWS_PT + j * 2 * CB
            pltpu.sync_copy(acc.at[pl.ds(base, 2 * CB)], g)
            pltpu.sync_copy(g, p_hbm.at[cid, pl.ds(base, 2 * CB)])
        tail = sid * ROWS_PT + (ROWS_PT // (2 * CB)) * 2 * CB
        pltpu.sync_copy(acc.at[pl.ds(tail, 104)], g.at[pl.ds(0, 104)])
        pltpu.sync_copy(g.at[pl.ds(0, 104)],
                        p_hbm.at[cid, pl.ds(tail, 104)])

        pltpu.sync_copy(cnt_s.at[pl.ds(sid * CNT_PT, CNT_PT)],
                        zcnt.at[pl.ds(0, CNT_PT)])

        @pl.when(cid == 0)
        def _():
            pltpu.sync_copy(zcnt.at[pl.ds(0, CNT_PT)],
                            cnt0_hbm.at[pl.ds(sid * CNT_PT, CNT_PT)])

        @pl.when(cid == 1)
        def _():
            pltpu.sync_copy(zcnt.at[pl.ds(0, CNT_PT)],
                            cnt1_hbm.at[pl.ds(sid * CNT_PT, CNT_PT)])


@functools.partial(
    pl.kernel,
    out_type=(
        jax.ShapeDtypeStruct((NC, N, D), jnp.float32),
        jax.ShapeDtypeStruct((N,), jnp.float32),
        jax.ShapeDtypeStruct((N,), jnp.float32),
    ),
    mesh=plsc.VectorSubcoreMesh(
        core_axis_name="c", subcore_axis_name="s", num_cores=NC, num_subcores=NS
    ),
    scratch_types=[
        pltpu.VMEM_SHARED((NA, D), jnp.float32),
        pltpu.VMEM_SHARED((NA,), jnp.float32),
        pltpu.VMEM((GC, 2 * CB), jnp.int32),
        pltpu.VMEM((GC, CB), jnp.int32),
        pltpu.VMEM((2 * CB, D), jnp.float32),
        pltpu.VMEM((CB, D), jnp.float32),
        pltpu.VMEM((CB, D), jnp.float32),
        pltpu.VMEM((CB,), jnp.float32),
        pltpu.VMEM((1008,), jnp.float32),
        pltpu.SemaphoreType.DMA,
        pltpu.SemaphoreType.DMA,
        pltpu.SemaphoreType.DMA,
        pltpu.SemaphoreType.DMA,
        pltpu.SemaphoreType.DMA,
    ],
)
def _edge_kernel(t_hbm, comb_hbm, dst_hbm, p_hbm, cnt0_hbm, cnt1_hbm,
                 *scratch):
    _edge_body(t_hbm, comb_hbm, dst_hbm, p_hbm, cnt0_hbm, cnt1_hbm,
               *scratch)


def _finalize_body(p_ref, c0_ref, c1_ref, o_ref):
    cnt = c0_ref[...] + c1_ref[...]
    inv = 1.0 / jnp.maximum(cnt, 1.0)
    o_ref[...] = (p_ref[0] + p_ref[1]) * inv[:, None]


def _finalize(p, cnt0, cnt1):
    return pl.pallas_call(
        _finalize_body,
        out_shape=jax.ShapeDtypeStruct((N, D), jnp.float32),
    )(p, cnt0, cnt1)


def _pad_edges(idx, pad_value):
    per_w = idx.reshape(NW, EPW)
    padded = jnp.pad(per_w, ((0, 0), (0, EPW_PAD - EPW)),
                     constant_values=pad_value)
    return padded.reshape(NW, NCHUNK, CB)


def kernel(x, edge_index, W, b):
    w1 = W[:, :D]
    w2 = W[:, D:]
    wa = (w1 - w2).T
    wb = w2.T
    b2d = b[None, :]
    x_pad = jnp.pad(x, ((0, NA - N), (0, 0)))
    table = _node_features(x_pad, wa, wb, b2d)
    dst_idx = _pad_edges(edge_index[1], N)
    src_idx = _pad_edges(edge_index[0], N)
    comb = jnp.concatenate([dst_idx, src_idx + NA], axis=-1)
    p, cnt0, cnt1 = _edge_kernel(table, comb, dst_idx)
    return _finalize(p, cnt0, cnt1)
